# Initial kernel scaffold; baseline (speedup 1.0000x reference)
import jax
import jax.numpy as jnp
from jax.experimental import pallas as pl


def kernel(pred_logits, pred_boxes):
    raise NotImplementedError("write your pallas kernel here")


# speedup vs baseline: 354.7110x; 354.7110x over previous
import functools

import jax
import jax.numpy as jnp
from jax import lax
from jax.experimental import pallas as pl
from jax.experimental.pallas import tpu as pltpu
from jax.experimental.pallas import tpu_sc as plsc

IMG = 640.0
IOU_THR = 0.01
SCORE_THR = 0.1
KEEP = 300
N = 5000
NEG = -1e9

NTILES = 16
T = 320
NPAD = NTILES * T
KPAD = 304


def _prep_body(lg_ref, bx_ref, mx_ref, lab_ref,
               x1_ref, y1_ref, x2_ref, y2_ref,
               x1o_ref, y1o_ref, x2o_ref, y2o_ref, ar_ref):
    lg = lg_ref[...]
    mx = jnp.max(lg, axis=-1)
    cls = lax.broadcasted_iota(jnp.int32, lg.shape, 2)
    lab = jnp.min(jnp.where(lg == mx[..., None], cls, lg.shape[-1]), axis=-1)
    labf = lab.astype(jnp.float32)

    cx = bx_ref[:, 0, :]
    cy = bx_ref[:, 1, :]
    w = bx_ref[:, 2, :]
    h = bx_ref[:, 3, :]
    x1 = (cx - 0.5 * w) * IMG
    y1 = (cy - 0.5 * h) * IMG
    x2 = (cx + 0.5 * w) * IMG
    y2 = (cy + 0.5 * h) * IMG
    off = labf * jnp.float32(2.0 * IMG + 1.0)
    x1o = x1 + off
    y1o = y1 + off
    x2o = x2 + off
    y2o = y2 + off
    ar = jnp.maximum(x2o - x1o, 0.0) * jnp.maximum(y2o - y1o, 0.0)

    mx_ref[...] = mx
    lab_ref[...] = labf
    x1_ref[...] = x1
    y1_ref[...] = y1
    x2_ref[...] = x2
    y2_ref[...] = y2
    x1o_ref[...] = x1o
    y1o_ref[...] = y1o
    x2o_ref[...] = x2o
    y2o_ref[...] = y2o
    ar_ref[...] = ar


_prep = pl.pallas_call(
    _prep_body,
    out_shape=[jax.ShapeDtypeStruct((2, N), jnp.float32)] * 11,
)


def _iota16():
    return lax.iota(jnp.int32, 16)


def _nms_body(s_hbm, x1o_hbm, y1o_hbm, x2o_hbm, y2o_hbm, ar_hbm,
              x1_hbm, y1_hbm, x2_hbm, y2_hbm, lab_hbm,
              os_hbm, ob_hbm, ol_hbm,
              stk, cand_v, all_v, os_v, ob_v, ol_v, shr):
    b = lax.axis_index("c")
    sid = lax.axis_index("s")
    base = b * NPAD + sid * T
    it = _iota16()
    zero16 = jnp.zeros((16,), jnp.float32)
    neg16 = jnp.full((16,), NEG, jnp.float32)

    srcs = (s_hbm, x1o_hbm, y1o_hbm, x2o_hbm, y2o_hbm, ar_hbm,
            x1_hbm, y1_hbm, x2_hbm, y2_hbm, lab_hbm)
    for r, src in enumerate(srcs):
        pltpu.sync_copy(src.at[pl.ds(base, T)], stk.at[pl.ds(r * T, T)])
    for c in range(T // 16):
        gi = (it + (base + c * 16)).astype(jnp.float32)
        stk[pl.ds(11 * T + c * 16, 16)] = gi
        for r in (12, 13, 14, 15):
            stk[pl.ds(r * T + c * 16, 16)] = zero16

    for c in range(KPAD // 16):
        os_v[pl.ds(c * 16, 16)] = zero16
        ol_v[pl.ds(c * 16, 16)] = jnp.full((16,), -1.0, jnp.float32)
    for c in range(KPAD * 4 // 16):
        ob_v[pl.ds(c * 16, 16)] = zero16

    def pick(k, carry):
        vm = neg16
        vi = jnp.zeros((16,), jnp.int32)
        for c in range(T // 16):
            v = stk[pl.ds(c * 16, 16)]
            gi = it + c * 16
            better = v > vm
            vm = jnp.where(better, v, vm)
            vi = jnp.where(better, gi, vi)
        m = jnp.max(vm)
        li = jnp.min(jnp.where(vm == m, vi, jnp.int32(1 << 30)))
        liv = jnp.full((16,), li, jnp.int32)
        cand = plsc.load_gather(stk, [it * T + liv])
        cand_v[...] = cand
        pltpu.sync_copy(cand_v, shr.at[pl.ds(sid * 16, 16)])
        plsc.subcore_barrier()
        pltpu.sync_copy(shr, all_v)
        plsc.subcore_barrier()

        sc16 = plsc.load_gather(all_v, [it * 16])
        gi16 = plsc.load_gather(all_v, [it * 16 + 11])
        wsc = jnp.max(sc16)
        gmin = jnp.min(jnp.where(sc16 == wsc, gi16, jnp.float32(1e9)))
        r16 = jnp.where((sc16 == wsc) & (gi16 == gmin), it, jnp.int32(16))
        r = jnp.min(r16)
        rv = jnp.full((16,), r * 16, jnp.int32)

        @pl.when(wsc > 0.0)
        def _do():
            wx1o = plsc.load_gather(all_v, [rv + 1])
            wy1o = plsc.load_gather(all_v, [rv + 2])
            wx2o = plsc.load_gather(all_v, [rv + 3])
            wy2o = plsc.load_gather(all_v, [rv + 4])
            war = plsc.load_gather(all_v, [rv + 5])
            for c in range(T // 16):
                x1c = stk[pl.ds(1 * T + c * 16, 16)]
                y1c = stk[pl.ds(2 * T + c * 16, 16)]
                x2c = stk[pl.ds(3 * T + c * 16, 16)]
                y2c = stk[pl.ds(4 * T + c * 16, 16)]
                arc = stk[pl.ds(5 * T + c * 16, 16)]
                sc = stk[pl.ds(c * 16, 16)]
                iw = jnp.maximum(jnp.minimum(x2c, wx2o) - jnp.maximum(x1c, wx1o), 0.0)
                ih = jnp.maximum(jnp.minimum(y2c, wy2o) - jnp.maximum(y1c, wy1o), 0.0)
                inter = iw * ih
                union = (war + arc) - inter
                iou = inter / jnp.maximum(union, jnp.float32(1e-9))
                stk[pl.ds(c * 16, 16)] = jnp.where(iou > jnp.float32(IOU_THR), neg16, sc)

            @pl.when(sid == 0)
            def _out():
                wscv = jnp.full((16,), wsc, jnp.float32)
                wlab = plsc.load_gather(all_v, [rv + 10])
                wbox = plsc.load_gather(all_v, [rv + jnp.minimum(it + 6, 15)])
                kv = jnp.full((16,), k, jnp.int32)
                lane0 = it == 0
                plsc.store_scatter(os_v, [kv], wscv, mask=lane0)
                plsc.store_scatter(ol_v, [kv], wlab, mask=lane0)
                plsc.store_scatter(ob_v, [kv * 4 + it], wbox, mask=it < 4)

        return carry

    lax.fori_loop(0, KEEP, pick, 0)

    @pl.when(sid == 0)
    def _flush():
        pltpu.sync_copy(os_v, os_hbm.at[pl.ds(b * KPAD, KPAD)])
        pltpu.sync_copy(ob_v, ob_hbm.at[pl.ds(b * KPAD * 4, KPAD * 4)])
        pltpu.sync_copy(ol_v, ol_hbm.at[pl.ds(b * KPAD, KPAD)])


_nms = pl.kernel(
    _nms_body,
    out_type=[
        jax.ShapeDtypeStruct((2 * KPAD,), jnp.float32),
        jax.ShapeDtypeStruct((2 * KPAD * 4,), jnp.float32),
        jax.ShapeDtypeStruct((2 * KPAD,), jnp.float32),
    ],
    mesh=plsc.VectorSubcoreMesh(core_axis_name="c", subcore_axis_name="s"),
    compiler_params=pltpu.CompilerParams(needs_layout_passes=False),
    scratch_types=[
        pltpu.VMEM((16 * T,), jnp.float32),
        pltpu.VMEM((16,), jnp.float32),
        pltpu.VMEM((256,), jnp.float32),
        pltpu.VMEM((KPAD,), jnp.float32),
        pltpu.VMEM((KPAD * 4,), jnp.float32),
        pltpu.VMEM((KPAD,), jnp.float32),
        pltpu.VMEM_SHARED((256,), jnp.float32),
    ],
)


def kernel(pred_logits, pred_boxes):
    boxes_t = jnp.transpose(pred_boxes, (0, 2, 1))
    (mx, labf, x1, y1, x2, y2,
     x1o, y1o, x2o, y2o, ar) = _prep(pred_logits, boxes_t)

    scores = jax.nn.sigmoid(mx)
    s = jnp.where(scores > SCORE_THR, scores, NEG)

    def pad(a, v):
        return jnp.pad(a, ((0, 0), (0, NPAD - N)), constant_values=v).reshape(-1)

    osc, obf, olf = _nms(
        pad(s, -1e9),
        pad(x1o, 0), pad(y1o, 0), pad(x2o, 0), pad(y2o, 0), pad(ar, 0),
        pad(x1, 0), pad(y1, 0), pad(x2, 0), pad(y2, 0), pad(labf, 0),
    )
    ob = obf.reshape(2, KPAD, 4)[:, :KEEP]
    return (ob, osc.reshape(2, KPAD)[:, :KEEP],
            olf.reshape(2, KPAD)[:, :KEEP].astype(jnp.int32))


# speedup vs baseline: 423.1783x; 1.1930x over previous
import functools

import jax
import jax.numpy as jnp
from jax import lax
from jax.experimental import pallas as pl
from jax.experimental.pallas import tpu as pltpu
from jax.experimental.pallas import tpu_sc as plsc

IMG = 640.0
IOU_THR = 0.01
SCORE_THR = 0.1
KEEP = 300
N = 5000
NEG = -1e9

NTILES = 16
T = 320
NPAD = NTILES * T
NFLAT = 2 * NPAD
KPAD = 304


def _prep_body(lg_ref, bx_ref, mx_ref, lab_ref,
               x1_ref, y1_ref, x2_ref, y2_ref,
               x1o_ref, y1o_ref, x2o_ref, y2o_ref, ar_ref):
    lg = lg_ref[...]
    mx = jnp.max(lg, axis=-1)
    cls = lax.broadcasted_iota(jnp.int32, lg.shape, 2)
    lab = jnp.min(jnp.where(lg == mx[..., None], cls, lg.shape[-1]), axis=-1)
    labf = lab.astype(jnp.float32)

    cx = bx_ref[:, 0, :]
    cy = bx_ref[:, 1, :]
    w = bx_ref[:, 2, :]
    h = bx_ref[:, 3, :]
    x1 = (cx - 0.5 * w) * IMG
    y1 = (cy - 0.5 * h) * IMG
    x2 = (cx + 0.5 * w) * IMG
    y2 = (cy + 0.5 * h) * IMG
    off = labf * jnp.float32(2.0 * IMG + 1.0)
    x1o = x1 + off
    y1o = y1 + off
    x2o = x2 + off
    y2o = y2 + off
    ar = jnp.maximum(x2o - x1o, 0.0) * jnp.maximum(y2o - y1o, 0.0)

    def emit(ref, val, fill):
        for b in range(2):
            ref[pl.ds(b * NPAD, N)] = val[b]
            ref[pl.ds(b * NPAD + N, NPAD - N)] = jnp.full(
                (NPAD - N,), fill, jnp.float32)

    emit(mx_ref, mx, NEG)
    emit(lab_ref, labf, 0.0)
    emit(x1_ref, x1, 0.0)
    emit(y1_ref, y1, 0.0)
    emit(x2_ref, x2, 0.0)
    emit(y2_ref, y2, 0.0)
    emit(x1o_ref, x1o, 0.0)
    emit(y1o_ref, y1o, 0.0)
    emit(x2o_ref, x2o, 0.0)
    emit(y2o_ref, y2o, 0.0)
    emit(ar_ref, ar, 0.0)


_prep = pl.pallas_call(
    _prep_body,
    out_shape=[jax.ShapeDtypeStruct((NFLAT,), jnp.float32)] * 11,
)


def _nms_body(s_hbm, x1o_hbm, y1o_hbm, x2o_hbm, y2o_hbm, ar_hbm,
              x1_hbm, y1_hbm, x2_hbm, y2_hbm, lab_hbm,
              os_hbm, ob_hbm, ol_hbm,
              stk, cand_v, all_v, os_v, ob_v, ol_v, shr):
    b = lax.axis_index("c")
    sid = lax.axis_index("s")
    base = b * NPAD + sid * T
    it = _iota16 = lax.iota(jnp.int32, 16)
    zero16 = jnp.zeros((16,), jnp.float32)
    neg16 = jnp.full((16,), NEG, jnp.float32)

    srcs = (s_hbm, x1o_hbm, y1o_hbm, x2o_hbm, y2o_hbm, ar_hbm,
            x1_hbm, y1_hbm, x2_hbm, y2_hbm, lab_hbm)
    for r, src in enumerate(srcs):
        pltpu.sync_copy(src.at[pl.ds(base, T)], stk.at[pl.ds(r * T, T)])
    for c in range(T // 16):
        gi = (it + (base + c * 16)).astype(jnp.float32)
        stk[pl.ds(11 * T + c * 16, 16)] = gi

    for c in range(KPAD // 16):
        os_v[pl.ds(c * 16, 16)] = zero16
        ol_v[pl.ds(c * 16, 16)] = jnp.full((16,), -1.0, jnp.float32)
    for c in range(KPAD * 4 // 16):
        ob_v[pl.ds(c * 16, 16)] = zero16

    vm0 = neg16
    vi0 = jnp.zeros((16,), jnp.int32)
    for c in range(T // 16):
        v = stk[pl.ds(c * 16, 16)]
        better = v > vm0
        vm0 = jnp.where(better, v, vm0)
        vi0 = jnp.where(better, it + c * 16, vi0)

    def pick(k, carry):
        vm, vi = carry
        m = jnp.max(vm)
        li = jnp.min(jnp.where(vm == m, vi, jnp.int32(1 << 30)))
        liv = jnp.full((16,), li, jnp.int32)
        cand = plsc.load_gather(stk, [it * T + liv])
        cand_v[...] = cand
        pbuf = (k % 2) * 256
        pltpu.sync_copy(cand_v, shr.at[pl.ds(pbuf + sid * 16, 16)])
        plsc.subcore_barrier()
        pltpu.sync_copy(shr.at[pl.ds(pbuf, 256)], all_v)

        sc16 = plsc.load_gather(all_v, [it * 16])
        gi16 = plsc.load_gather(all_v, [it * 16 + 11])
        wsc = jnp.max(sc16)
        gmin = jnp.min(jnp.where(sc16 == wsc, gi16, jnp.float32(1e9)))
        r16 = jnp.where((sc16 == wsc) & (gi16 == gmin), it, jnp.int32(16))
        rv = jnp.full((16,), jnp.min(r16) * 16, jnp.int32)

        wx1o = plsc.load_gather(all_v, [rv + 1])
        wy1o = plsc.load_gather(all_v, [rv + 2])
        wx2o = plsc.load_gather(all_v, [rv + 3])
        wy2o = plsc.load_gather(all_v, [rv + 4])
        war = plsc.load_gather(all_v, [rv + 5])

        nvm = neg16
        nvi = jnp.zeros((16,), jnp.int32)
        for c in range(T // 16):
            x1c = stk[pl.ds(1 * T + c * 16, 16)]
            y1c = stk[pl.ds(2 * T + c * 16, 16)]
            x2c = stk[pl.ds(3 * T + c * 16, 16)]
            y2c = stk[pl.ds(4 * T + c * 16, 16)]
            arc = stk[pl.ds(5 * T + c * 16, 16)]
            sc = stk[pl.ds(c * 16, 16)]
            iw = jnp.maximum(jnp.minimum(x2c, wx2o) - jnp.maximum(x1c, wx1o), 0.0)
            ih = jnp.maximum(jnp.minimum(y2c, wy2o) - jnp.maximum(y1c, wy1o), 0.0)
            inter = iw * ih
            union = (war + arc) - inter
            iou = inter / jnp.maximum(union, jnp.float32(1e-9))
            snew = jnp.where(iou > jnp.float32(IOU_THR), neg16, sc)
            stk[pl.ds(c * 16, 16)] = snew
            better = snew > nvm
            nvm = jnp.where(better, snew, nvm)
            nvi = jnp.where(better, it + c * 16, nvi)

        wscv = jnp.full((16,), wsc, jnp.float32)
        ok = wscv > 0.0
        wlab = plsc.load_gather(all_v, [rv + 10])
        wbox = plsc.load_gather(all_v, [rv + jnp.minimum(it + 6, 15)])
        kv = jnp.full((16,), k, jnp.int32)
        lane0 = (it == 0) & ok
        plsc.store_scatter(os_v, [kv], wscv, mask=lane0)
        plsc.store_scatter(ol_v, [kv], wlab, mask=lane0)
        plsc.store_scatter(ob_v, [kv * 4 + it], wbox, mask=(it < 4) & ok)

        return nvm, nvi

    lax.fori_loop(0, KEEP, pick, (vm0, vi0))

    @pl.when(sid == 0)
    def _flush():
        pltpu.sync_copy(os_v, os_hbm.at[pl.ds(b * KPAD, KPAD)])
        pltpu.sync_copy(ob_v, ob_hbm.at[pl.ds(b * KPAD * 4, KPAD * 4)])
        pltpu.sync_copy(ol_v, ol_hbm.at[pl.ds(b * KPAD, KPAD)])


_nms = pl.kernel(
    _nms_body,
    out_type=[
        jax.ShapeDtypeStruct((2 * KPAD,), jnp.float32),
        jax.ShapeDtypeStruct((2 * KPAD * 4,), jnp.float32),
        jax.ShapeDtypeStruct((2 * KPAD,), jnp.float32),
    ],
    mesh=plsc.VectorSubcoreMesh(core_axis_name="c", subcore_axis_name="s"),
    compiler_params=pltpu.CompilerParams(needs_layout_passes=False),
    scratch_types=[
        pltpu.VMEM((16 * T,), jnp.float32),
        pltpu.VMEM((16,), jnp.float32),
        pltpu.VMEM((256,), jnp.float32),
        pltpu.VMEM((KPAD,), jnp.float32),
        pltpu.VMEM((KPAD * 4,), jnp.float32),
        pltpu.VMEM((KPAD,), jnp.float32),
        pltpu.VMEM_SHARED((512,), jnp.float32),
    ],
)


def kernel(pred_logits, pred_boxes):
    boxes_t = jnp.transpose(pred_boxes, (0, 2, 1))
    (mx, labf, x1, y1, x2, y2,
     x1o, y1o, x2o, y2o, ar) = _prep(pred_logits, boxes_t)

    scores = jax.nn.sigmoid(mx)
    s = jnp.where(scores > SCORE_THR, scores, NEG)

    osc, obf, olf = _nms(s, x1o, y1o, x2o, y2o, ar, x1, y1, x2, y2, labf)
    ob = obf.reshape(2, KPAD, 4)[:, :KEEP]
    return (ob, osc.reshape(2, KPAD)[:, :KEEP],
            olf.reshape(2, KPAD)[:, :KEEP].astype(jnp.int32))


# speedup vs baseline: 511.2980x; 1.2082x over previous
import functools

import jax
import jax.numpy as jnp
from jax import lax
from jax.experimental import pallas as pl
from jax.experimental.pallas import tpu as pltpu
from jax.experimental.pallas import tpu_sc as plsc

IMG = 640.0
IOU_THR = 0.01
SCORE_THR = 0.1
KEEP = 300
N = 5000
NEG = -1e9

NTILES = 16
T = 320
NPAD = NTILES * T
NFLAT = 2 * NPAD
KPAD = 304


def _prep_body(lg_ref, bx_ref, mx_ref, lab_ref,
               x1_ref, y1_ref, x2_ref, y2_ref,
               x1o_ref, y1o_ref, x2o_ref, y2o_ref, ar_ref):
    lg = lg_ref[...]
    mx = jnp.max(lg, axis=-1)
    cls = lax.broadcasted_iota(jnp.int32, lg.shape, 2)
    lab = jnp.min(jnp.where(lg == mx[..., None], cls, lg.shape[-1]), axis=-1)
    labf = lab.astype(jnp.float32)

    cx = bx_ref[:, 0, :]
    cy = bx_ref[:, 1, :]
    w = bx_ref[:, 2, :]
    h = bx_ref[:, 3, :]
    x1 = (cx - 0.5 * w) * IMG
    y1 = (cy - 0.5 * h) * IMG
    x2 = (cx + 0.5 * w) * IMG
    y2 = (cy + 0.5 * h) * IMG
    off = labf * jnp.float32(2.0 * IMG + 1.0)
    x1o = x1 + off
    y1o = y1 + off
    x2o = x2 + off
    y2o = y2 + off
    ar = jnp.maximum(x2o - x1o, 0.0) * jnp.maximum(y2o - y1o, 0.0)

    def emit(ref, val, fill):
        for b in range(2):
            ref[pl.ds(b * NPAD, N)] = val[b]
            ref[pl.ds(b * NPAD + N, NPAD - N)] = jnp.full(
                (NPAD - N,), fill, jnp.float32)

    emit(mx_ref, mx, NEG)
    emit(lab_ref, labf, 0.0)
    emit(x1_ref, x1, 0.0)
    emit(y1_ref, y1, 0.0)
    emit(x2_ref, x2, 0.0)
    emit(y2_ref, y2, 0.0)
    emit(x1o_ref, x1o, 0.0)
    emit(y1o_ref, y1o, 0.0)
    emit(x2o_ref, x2o, 0.0)
    emit(y2o_ref, y2o, 0.0)
    emit(ar_ref, ar, 0.0)


_prep = pl.pallas_call(
    _prep_body,
    out_shape=[jax.ShapeDtypeStruct((NFLAT,), jnp.float32)] * 11,
)


def _nms_body(s_hbm, x1o_hbm, y1o_hbm, x2o_hbm, y2o_hbm, ar_hbm,
              x1_hbm, y1_hbm, x2_hbm, y2_hbm, lab_hbm,
              os_hbm, ob_hbm, ol_hbm,
              stk, cand_v, all_v, os_v, ob_v, ol_v, shr):
    b = lax.axis_index("c")
    sid = lax.axis_index("s")
    base = b * NPAD + sid * T
    it = _iota16 = lax.iota(jnp.int32, 16)
    zero16 = jnp.zeros((16,), jnp.float32)
    neg16 = jnp.full((16,), NEG, jnp.float32)

    srcs = (s_hbm, x1o_hbm, y1o_hbm, x2o_hbm, y2o_hbm, ar_hbm,
            x1_hbm, y1_hbm, x2_hbm, y2_hbm, lab_hbm)
    for r, src in enumerate(srcs):
        pltpu.sync_copy(src.at[pl.ds(base, T)], stk.at[pl.ds(r * T, T)])
    for c in range(T // 16):
        gi = (it + (base + c * 16)).astype(jnp.float32)
        stk[pl.ds(11 * T + c * 16, 16)] = gi

    for c in range(KPAD // 16):
        os_v[pl.ds(c * 16, 16)] = zero16
        ol_v[pl.ds(c * 16, 16)] = jnp.full((16,), -1.0, jnp.float32)
    for c in range(KPAD * 4 // 16):
        ob_v[pl.ds(c * 16, 16)] = zero16

    vm1_0 = neg16
    vi1_0 = jnp.zeros((16,), jnp.int32)
    vm2_0 = neg16
    vi2_0 = jnp.zeros((16,), jnp.int32)
    for c in range(T // 16):
        v = stk[pl.ds(c * 16, 16)]
        gi_c = it + c * 16
        gt1 = v > vm1_0
        gt2 = v > vm2_0
        vm2_0 = jnp.where(gt1, vm1_0, jnp.where(gt2, v, vm2_0))
        vi2_0 = jnp.where(gt1, vi1_0, jnp.where(gt2, gi_c, vi2_0))
        vm1_0 = jnp.where(gt1, v, vm1_0)
        vi1_0 = jnp.where(gt1, gi_c, vi1_0)

    BIG = jnp.int32(1 << 30)

    def cond(carry):
        vm1, vi1, vm2, vi2, kk, go, rr = carry
        return (kk < KEEP) & (go > 0.0)

    def round_body(carry):
        vm1, vi1, vm2, vi2, kk, go, rr = carry
        m1 = jnp.max(vm1)
        li1 = jnp.min(jnp.where(vm1 == m1, vi1, BIG))
        is1 = (vm1 == m1) & (vi1 == li1)
        vsec = jnp.where(is1, vm2, vm1)
        isec = jnp.where(is1, vi2, vi1)
        m2 = jnp.max(vsec)
        li2 = jnp.min(jnp.where(vsec == m2, isec, BIG))
        cand1 = plsc.load_gather(stk, [it * T + jnp.full((16,), li1, jnp.int32)])
        cand2 = plsc.load_gather(stk, [it * T + jnp.full((16,), li2, jnp.int32)])
        cand_v[pl.ds(0, 16)] = cand1
        cand_v[pl.ds(16, 16)] = cand2
        pbuf = (rr % 2) * 512
        pltpu.sync_copy(cand_v, shr.at[pl.ds(pbuf + sid * 32, 32)])
        plsc.subcore_barrier()
        pltpu.sync_copy(shr.at[pl.ds(pbuf, 512)], all_v)

        sA = plsc.load_gather(all_v, [it * 32])
        sB = plsc.load_gather(all_v, [it * 32 + 16])
        w1s = jnp.max(sA)
        r1 = jnp.min(jnp.where(sA == w1s, it, jnp.int32(16)))
        r1v = jnp.full((16,), r1, jnp.int32)
        rv1 = r1v * 32
        isr1 = it == r1v
        s2c = jnp.where(isr1, sB, sA)
        w2s = jnp.max(s2c)
        l2 = jnp.min(jnp.where(s2c == w2s, it, jnp.int32(16)))
        rv2s = l2 * 32 + jnp.where(l2 == r1, 16, 0)
        rv2 = jnp.full((16,), rv2s, jnp.int32)

        wx1o = plsc.load_gather(all_v, [rv1 + 1])
        wy1o = plsc.load_gather(all_v, [rv1 + 2])
        wx2o = plsc.load_gather(all_v, [rv1 + 3])
        wy2o = plsc.load_gather(all_v, [rv1 + 4])
        war = plsc.load_gather(all_v, [rv1 + 5])
        ux1o = plsc.load_gather(all_v, [rv2 + 1])
        uy1o = plsc.load_gather(all_v, [rv2 + 2])
        ux2o = plsc.load_gather(all_v, [rv2 + 3])
        uy2o = plsc.load_gather(all_v, [rv2 + 4])
        uar = plsc.load_gather(all_v, [rv2 + 5])

        iw12 = jnp.maximum(jnp.minimum(wx2o, ux2o) - jnp.maximum(wx1o, ux1o), 0.0)
        ih12 = jnp.maximum(jnp.minimum(wy2o, uy2o) - jnp.maximum(wy1o, uy1o), 0.0)
        in12 = iw12 * ih12
        un12 = (war + uar) - in12
        iou12 = in12 / jnp.maximum(un12, jnp.float32(1e-9))
        w2sv = jnp.full((16,), w2s, jnp.float32)
        valid2v = (w2sv > 0.0) & jnp.logical_not(iou12 > jnp.float32(IOU_THR))
        ux1o = jnp.where(valid2v, ux1o, 0.0)
        uy1o = jnp.where(valid2v, uy1o, 0.0)
        ux2o = jnp.where(valid2v, ux2o, 0.0)
        uy2o = jnp.where(valid2v, uy2o, 0.0)
        uar = jnp.where(valid2v, uar, 0.0)

        nvm1 = neg16
        nvi1 = jnp.zeros((16,), jnp.int32)
        nvm2 = neg16
        nvi2 = jnp.zeros((16,), jnp.int32)
        for c in range(T // 16):
            x1c = stk[pl.ds(1 * T + c * 16, 16)]
            y1c = stk[pl.ds(2 * T + c * 16, 16)]
            x2c = stk[pl.ds(3 * T + c * 16, 16)]
            y2c = stk[pl.ds(4 * T + c * 16, 16)]
            arc = stk[pl.ds(5 * T + c * 16, 16)]
            sc = stk[pl.ds(c * 16, 16)]
            iw = jnp.maximum(jnp.minimum(x2c, wx2o) - jnp.maximum(x1c, wx1o), 0.0)
            ih = jnp.maximum(jnp.minimum(y2c, wy2o) - jnp.maximum(y1c, wy1o), 0.0)
            inter = iw * ih
            union = (war + arc) - inter
            iou = inter / jnp.maximum(union, jnp.float32(1e-9))
            iw2 = jnp.maximum(jnp.minimum(x2c, ux2o) - jnp.maximum(x1c, ux1o), 0.0)
            ih2 = jnp.maximum(jnp.minimum(y2c, uy2o) - jnp.maximum(y1c, uy1o), 0.0)
            inter2 = iw2 * ih2
            union2 = (uar + arc) - inter2
            iou2 = inter2 / jnp.maximum(union2, jnp.float32(1e-9))
            kill = (iou > jnp.float32(IOU_THR)) | (iou2 > jnp.float32(IOU_THR))
            snew = jnp.where(kill, neg16, sc)
            stk[pl.ds(c * 16, 16)] = snew
            gi_c = it + c * 16
            gt1 = snew > nvm1
            gt2 = snew > nvm2
            nvm2 = jnp.where(gt1, nvm1, jnp.where(gt2, snew, nvm2))
            nvi2 = jnp.where(gt1, nvi1, jnp.where(gt2, gi_c, nvi2))
            nvm1 = jnp.where(gt1, snew, nvm1)
            nvi1 = jnp.where(gt1, gi_c, nvi1)

        w1sv = jnp.full((16,), w1s, jnp.float32)
        ok1v = w1sv > 0.0
        wlab = plsc.load_gather(all_v, [rv1 + 10])
        wbox = plsc.load_gather(all_v, [rv1 + jnp.minimum(it + 6, 15)])
        ulab = plsc.load_gather(all_v, [rv2 + 10])
        ubox = plsc.load_gather(all_v, [rv2 + jnp.minimum(it + 6, 15)])
        kv = jnp.full((16,), kk, jnp.int32)
        kv2 = kv + 1
        lane0 = it == 0
        ok2v = valid2v & ok1v
        plsc.store_scatter(os_v, [kv], w1sv, mask=lane0 & ok1v)
        plsc.store_scatter(ol_v, [kv], wlab, mask=lane0 & ok1v)
        plsc.store_scatter(ob_v, [kv * 4 + it], wbox, mask=(it < 4) & ok1v)
        plsc.store_scatter(os_v, [kv2], w2sv, mask=lane0 & ok2v)
        plsc.store_scatter(ol_v, [kv2], ulab, mask=lane0 & ok2v)
        plsc.store_scatter(ob_v, [kv2 * 4 + it], ubox, mask=(it < 4) & ok2v)

        ok1 = (w1s > 0.0).astype(jnp.int32)
        ok2 = jnp.max(jnp.where(ok2v, 1, 0))
        kk_next = kk + ok1 + ok1 * ok2
        return nvm1, nvi1, nvm2, nvi2, kk_next, w1s, rr + 1

    lax.while_loop(cond, round_body,
                   (vm1_0, vi1_0, vm2_0, vi2_0,
                    jnp.int32(0), jnp.float32(1.0), jnp.int32(0)))

    @pl.when(sid == 0)
    def _flush():
        pltpu.sync_copy(os_v, os_hbm.at[pl.ds(b * KPAD, KPAD)])
        pltpu.sync_copy(ob_v, ob_hbm.at[pl.ds(b * KPAD * 4, KPAD * 4)])
        pltpu.sync_copy(ol_v, ol_hbm.at[pl.ds(b * KPAD, KPAD)])


_nms = pl.kernel(
    _nms_body,
    out_type=[
        jax.ShapeDtypeStruct((2 * KPAD,), jnp.float32),
        jax.ShapeDtypeStruct((2 * KPAD * 4,), jnp.float32),
        jax.ShapeDtypeStruct((2 * KPAD,), jnp.float32),
    ],
    mesh=plsc.VectorSubcoreMesh(core_axis_name="c", subcore_axis_name="s"),
    compiler_params=pltpu.CompilerParams(needs_layout_passes=False),
    scratch_types=[
        pltpu.VMEM((16 * T,), jnp.float32),
        pltpu.VMEM((32,), jnp.float32),
        pltpu.VMEM((512,), jnp.float32),
        pltpu.VMEM((KPAD,), jnp.float32),
        pltpu.VMEM((KPAD * 4,), jnp.float32),
        pltpu.VMEM((KPAD,), jnp.float32),
        pltpu.VMEM_SHARED((1024,), jnp.float32),
    ],
)


def kernel(pred_logits, pred_boxes):
    boxes_t = jnp.transpose(pred_boxes, (0, 2, 1))
    (mx, labf, x1, y1, x2, y2,
     x1o, y1o, x2o, y2o, ar) = _prep(pred_logits, boxes_t)

    scores = jax.nn.sigmoid(mx)
    s = jnp.where(scores > SCORE_THR, scores, NEG)

    osc, obf, olf = _nms(s, x1o, y1o, x2o, y2o, ar, x1, y1, x2, y2, labf)
    ob = obf.reshape(2, KPAD, 4)[:, :KEEP]
    return (ob, osc.reshape(2, KPAD)[:, :KEEP],
            olf.reshape(2, KPAD)[:, :KEEP].astype(jnp.int32))


# speedup vs baseline: 549.2434x; 1.0742x over previous
import functools

import jax
import jax.numpy as jnp
from jax import lax
from jax.experimental import pallas as pl
from jax.experimental.pallas import tpu as pltpu
from jax.experimental.pallas import tpu_sc as plsc

IMG = 640.0
IOU_THR = 0.01
SCORE_THR = 0.1
KEEP = 300
N = 5000
NEG = -1e9

NTILES = 16
T = 320
NPAD = NTILES * T
NFLAT = 2 * NPAD
KPAD = 304


def _prep_body(lg_ref, bx_ref, mx_ref, lab_ref,
               x1_ref, y1_ref, x2_ref, y2_ref,
               x1o_ref, y1o_ref, x2o_ref, y2o_ref, ar_ref):
    lg = lg_ref[...]
    mx = jnp.max(lg, axis=0)
    cls = lax.broadcasted_iota(jnp.int32, lg.shape, 0)
    lab = jnp.min(jnp.where(lg == mx[None, :], cls, lg.shape[0]), axis=0)
    labf = lab.astype(jnp.float32)

    cx = bx_ref[0]
    cy = bx_ref[1]
    w = bx_ref[2]
    h = bx_ref[3]
    x1 = (cx - 0.5 * w) * IMG
    y1 = (cy - 0.5 * h) * IMG
    x2 = (cx + 0.5 * w) * IMG
    y2 = (cy + 0.5 * h) * IMG
    off = labf * jnp.float32(2.0 * IMG + 1.0)
    x1o = x1 + off
    y1o = y1 + off
    x2o = x2 + off
    y2o = y2 + off
    ar = jnp.maximum(x2o - x1o, 0.0) * jnp.maximum(y2o - y1o, 0.0)

    def emit(ref, val, fill):
        for b in range(2):
            ref[pl.ds(b * NPAD, N)] = val[b * N:(b + 1) * N]
            ref[pl.ds(b * NPAD + N, NPAD - N)] = jnp.full(
                (NPAD - N,), fill, jnp.float32)

    emit(mx_ref, mx, NEG)
    emit(lab_ref, labf, 0.0)
    emit(x1_ref, x1, 0.0)
    emit(y1_ref, y1, 0.0)
    emit(x2_ref, x2, 0.0)
    emit(y2_ref, y2, 0.0)
    emit(x1o_ref, x1o, 0.0)
    emit(y1o_ref, y1o, 0.0)
    emit(x2o_ref, x2o, 0.0)
    emit(y2o_ref, y2o, 0.0)
    emit(ar_ref, ar, 0.0)


_prep = pl.pallas_call(
    _prep_body,
    out_shape=[jax.ShapeDtypeStruct((NFLAT,), jnp.float32)] * 11,
)


def _nms_body(s_hbm, x1o_hbm, y1o_hbm, x2o_hbm, y2o_hbm, ar_hbm,
              x1_hbm, y1_hbm, x2_hbm, y2_hbm, lab_hbm,
              os_hbm, ob_hbm, ol_hbm,
              stk, cand_v, all_v, os_v, ob_v, ol_v, shr):
    b = lax.axis_index("c")
    sid = lax.axis_index("s")
    base = b * NPAD + sid * T
    it = _iota16 = lax.iota(jnp.int32, 16)
    zero16 = jnp.zeros((16,), jnp.float32)
    neg16 = jnp.full((16,), NEG, jnp.float32)

    srcs = (s_hbm, x1o_hbm, y1o_hbm, x2o_hbm, y2o_hbm, ar_hbm,
            x1_hbm, y1_hbm, x2_hbm, y2_hbm, lab_hbm)
    for r, src in enumerate(srcs):
        pltpu.sync_copy(src.at[pl.ds(base, T)], stk.at[pl.ds(r * T, T)])
    for c in range(T // 16):
        gi = (it + (base + c * 16)).astype(jnp.float32)
        stk[pl.ds(11 * T + c * 16, 16)] = gi

    for c in range(KPAD // 16):
        os_v[pl.ds(c * 16, 16)] = zero16
        ol_v[pl.ds(c * 16, 16)] = jnp.full((16,), -1.0, jnp.float32)
    for c in range(KPAD * 4 // 16):
        ob_v[pl.ds(c * 16, 16)] = zero16

    vm1_0 = neg16
    vi1_0 = jnp.zeros((16,), jnp.int32)
    vm2_0 = neg16
    vi2_0 = jnp.zeros((16,), jnp.int32)
    for c in range(T // 16):
        v = stk[pl.ds(c * 16, 16)]
        gi_c = it + c * 16
        gt1 = v > vm1_0
        gt2 = v > vm2_0
        vm2_0 = jnp.where(gt1, vm1_0, jnp.where(gt2, v, vm2_0))
        vi2_0 = jnp.where(gt1, vi1_0, jnp.where(gt2, gi_c, vi2_0))
        vm1_0 = jnp.where(gt1, v, vm1_0)
        vi1_0 = jnp.where(gt1, gi_c, vi1_0)

    BIG = jnp.int32(1 << 30)

    def cond(carry):
        vm1, vi1, vm2, vi2, kk, go, rr = carry
        return (kk < KEEP) & (go > 0.0)

    def round_body(carry):
        vm1, vi1, vm2, vi2, kk, go, rr = carry
        m1 = jnp.max(vm1)
        li1 = jnp.min(jnp.where(vm1 == m1, vi1, BIG))
        is1 = (vm1 == m1) & (vi1 == li1)
        vsec = jnp.where(is1, vm2, vm1)
        isec = jnp.where(is1, vi2, vi1)
        m2 = jnp.max(vsec)
        li2 = jnp.min(jnp.where(vsec == m2, isec, BIG))
        cand1 = plsc.load_gather(stk, [it * T + jnp.full((16,), li1, jnp.int32)])
        cand2 = plsc.load_gather(stk, [it * T + jnp.full((16,), li2, jnp.int32)])
        cand_v[pl.ds(0, 16)] = cand1
        cand_v[pl.ds(16, 16)] = cand2
        pbuf = (rr % 2) * 512
        pltpu.sync_copy(cand_v, shr.at[pl.ds(pbuf + sid * 32, 32)])
        plsc.subcore_barrier()
        pltpu.sync_copy(shr.at[pl.ds(pbuf, 512)], all_v)

        sA = plsc.load_gather(all_v, [it * 32])
        sB = plsc.load_gather(all_v, [it * 32 + 16])
        w1s = jnp.max(sA)
        r1 = jnp.min(jnp.where(sA == w1s, it, jnp.int32(16)))
        r1v = jnp.full((16,), r1, jnp.int32)
        rv1 = r1v * 32
        isr1 = it == r1v
        s2c = jnp.where(isr1, sB, sA)
        w2s = jnp.max(s2c)
        l2 = jnp.min(jnp.where(s2c == w2s, it, jnp.int32(16)))
        rv2s = l2 * 32 + jnp.where(l2 == r1, 16, 0)
        rv2 = jnp.full((16,), rv2s, jnp.int32)

        wx1o = plsc.load_gather(all_v, [rv1 + 1])
        wy1o = plsc.load_gather(all_v, [rv1 + 2])
        wx2o = plsc.load_gather(all_v, [rv1 + 3])
        wy2o = plsc.load_gather(all_v, [rv1 + 4])
        war = plsc.load_gather(all_v, [rv1 + 5])
        ux1o = plsc.load_gather(all_v, [rv2 + 1])
        uy1o = plsc.load_gather(all_v, [rv2 + 2])
        ux2o = plsc.load_gather(all_v, [rv2 + 3])
        uy2o = plsc.load_gather(all_v, [rv2 + 4])
        uar = plsc.load_gather(all_v, [rv2 + 5])

        iw12 = jnp.maximum(jnp.minimum(wx2o, ux2o) - jnp.maximum(wx1o, ux1o), 0.0)
        ih12 = jnp.maximum(jnp.minimum(wy2o, uy2o) - jnp.maximum(wy1o, uy1o), 0.0)
        in12 = iw12 * ih12
        un12 = (war + uar) - in12
        iou12 = in12 / jnp.maximum(un12, jnp.float32(1e-9))
        w2sv = jnp.full((16,), w2s, jnp.float32)
        valid2v = (w2sv > 0.0) & jnp.logical_not(iou12 > jnp.float32(IOU_THR))
        ux1o = jnp.where(valid2v, ux1o, 0.0)
        uy1o = jnp.where(valid2v, uy1o, 0.0)
        ux2o = jnp.where(valid2v, ux2o, 0.0)
        uy2o = jnp.where(valid2v, uy2o, 0.0)
        uar = jnp.where(valid2v, uar, 0.0)

        nvm1 = neg16
        nvi1 = jnp.zeros((16,), jnp.int32)
        nvm2 = neg16
        nvi2 = jnp.zeros((16,), jnp.int32)
        for c in range(T // 16):
            x1c = stk[pl.ds(1 * T + c * 16, 16)]
            y1c = stk[pl.ds(2 * T + c * 16, 16)]
            x2c = stk[pl.ds(3 * T + c * 16, 16)]
            y2c = stk[pl.ds(4 * T + c * 16, 16)]
            arc = stk[pl.ds(5 * T + c * 16, 16)]
            sc = stk[pl.ds(c * 16, 16)]
            iw = jnp.maximum(jnp.minimum(x2c, wx2o) - jnp.maximum(x1c, wx1o), 0.0)
            ih = jnp.maximum(jnp.minimum(y2c, wy2o) - jnp.maximum(y1c, wy1o), 0.0)
            inter = iw * ih
            union = (war + arc) - inter
            iou = inter / jnp.maximum(union, jnp.float32(1e-9))
            iw2 = jnp.maximum(jnp.minimum(x2c, ux2o) - jnp.maximum(x1c, ux1o), 0.0)
            ih2 = jnp.maximum(jnp.minimum(y2c, uy2o) - jnp.maximum(y1c, uy1o), 0.0)
            inter2 = iw2 * ih2
            union2 = (uar + arc) - inter2
            iou2 = inter2 / jnp.maximum(union2, jnp.float32(1e-9))
            kill = (iou > jnp.float32(IOU_THR)) | (iou2 > jnp.float32(IOU_THR))
            snew = jnp.where(kill, neg16, sc)
            stk[pl.ds(c * 16, 16)] = snew
            gi_c = it + c * 16
            gt1 = snew > nvm1
            gt2 = snew > nvm2
            nvm2 = jnp.where(gt1, nvm1, jnp.where(gt2, snew, nvm2))
            nvi2 = jnp.where(gt1, nvi1, jnp.where(gt2, gi_c, nvi2))
            nvm1 = jnp.where(gt1, snew, nvm1)
            nvi1 = jnp.where(gt1, gi_c, nvi1)

        w1sv = jnp.full((16,), w1s, jnp.float32)
        ok1v = w1sv > 0.0
        wlab = plsc.load_gather(all_v, [rv1 + 10])
        wbox = plsc.load_gather(all_v, [rv1 + jnp.minimum(it + 6, 15)])
        ulab = plsc.load_gather(all_v, [rv2 + 10])
        ubox = plsc.load_gather(all_v, [rv2 + jnp.minimum(it + 6, 15)])
        kv = jnp.full((16,), kk, jnp.int32)
        kv2 = kv + 1
        lane0 = it == 0
        ok2v = valid2v & ok1v
        plsc.store_scatter(os_v, [kv], w1sv, mask=lane0 & ok1v)
        plsc.store_scatter(ol_v, [kv], wlab, mask=lane0 & ok1v)
        plsc.store_scatter(ob_v, [kv * 4 + it], wbox, mask=(it < 4) & ok1v)
        plsc.store_scatter(os_v, [kv2], w2sv, mask=lane0 & ok2v)
        plsc.store_scatter(ol_v, [kv2], ulab, mask=lane0 & ok2v)
        plsc.store_scatter(ob_v, [kv2 * 4 + it], ubox, mask=(it < 4) & ok2v)

        ok1 = (w1s > 0.0).astype(jnp.int32)
        ok2 = jnp.max(jnp.where(ok2v, 1, 0))
        kk_next = kk + ok1 + ok1 * ok2
        return nvm1, nvi1, nvm2, nvi2, kk_next, w1s, rr + 1

    lax.while_loop(cond, round_body,
                   (vm1_0, vi1_0, vm2_0, vi2_0,
                    jnp.int32(0), jnp.float32(1.0), jnp.int32(0)))

    @pl.when(sid == 0)
    def _flush():
        pltpu.sync_copy(os_v, os_hbm.at[pl.ds(b * KPAD, KPAD)])
        pltpu.sync_copy(ob_v, ob_hbm.at[pl.ds(b * KPAD * 4, KPAD * 4)])
        pltpu.sync_copy(ol_v, ol_hbm.at[pl.ds(b * KPAD, KPAD)])


_nms = pl.kernel(
    _nms_body,
    out_type=[
        jax.ShapeDtypeStruct((2 * KPAD,), jnp.float32),
        jax.ShapeDtypeStruct((2 * KPAD * 4,), jnp.float32),
        jax.ShapeDtypeStruct((2 * KPAD,), jnp.float32),
    ],
    mesh=plsc.VectorSubcoreMesh(core_axis_name="c", subcore_axis_name="s"),
    compiler_params=pltpu.CompilerParams(needs_layout_passes=False),
    scratch_types=[
        pltpu.VMEM((16 * T,), jnp.float32),
        pltpu.VMEM((32,), jnp.float32),
        pltpu.VMEM((512,), jnp.float32),
        pltpu.VMEM((KPAD,), jnp.float32),
        pltpu.VMEM((KPAD * 4,), jnp.float32),
        pltpu.VMEM((KPAD,), jnp.float32),
        pltpu.VMEM_SHARED((1024,), jnp.float32),
    ],
)


def kernel(pred_logits, pred_boxes):
    logits_t = jnp.transpose(pred_logits, (2, 0, 1)).reshape(80, 2 * N)
    boxes_t = jnp.transpose(pred_boxes, (2, 0, 1)).reshape(4, 2 * N)
    (mx, labf, x1, y1, x2, y2,
     x1o, y1o, x2o, y2o, ar) = _prep(logits_t, boxes_t)

    scores = jax.nn.sigmoid(mx)
    s = jnp.where(scores > SCORE_THR, scores, NEG)

    osc, obf, olf = _nms(s, x1o, y1o, x2o, y2o, ar, x1, y1, x2, y2, labf)
    ob = obf.reshape(2, KPAD, 4)[:, :KEEP]
    return (ob, osc.reshape(2, KPAD)[:, :KEEP],
            olf.reshape(2, KPAD)[:, :KEEP].astype(jnp.int32))
